# Initial kernel scaffold; baseline (speedup 1.0000x reference)
import jax
import jax.numpy as jnp
from jax.experimental import pallas as pl


def kernel(x, edge_index, batch, W_l0, b_l0, W_r0, W_l1, b_l1, W_r1, W_c1, b_c1, W_c2, b_c2):
    raise NotImplementedError("write your pallas kernel here")


# speedup vs baseline: 3.1604x; 3.1604x over previous
import functools

import jax
import jax.numpy as jnp
from jax import lax
from jax.experimental import pallas as pl
from jax.experimental.pallas import tpu as pltpu
from jax.experimental.pallas import tpu_sc as plsc

N, E, D, H, G = 10000, 320000, 128, 128, 64
NC, NS = 2, 16
NW = NC * NS
C = 128
NCH = 80
GC = 16
EPT = NCH * C
EP = EPT * NW
NP = 10240
RPT = NP // NS
TRASH = N
R = 400
NBLK = N // R


def _sc_agg_body(x_hbm, src_hbm, dst_hbm, acc_hbm, acc_sh, rows, idx_s, idx_d):
    cid = lax.axis_index("c")
    sid = lax.axis_index("s")
    wid = cid * NS + sid

    @pl.loop(0, C)
    def _(r):
        @pl.loop(0, D // 16)
        def _(c):
            rows.at[r, pl.ds(c * 16, 16)][...] = jnp.zeros((16,), jnp.float32)

    @pl.loop(0, RPT // C)
    def _(k):
        pltpu.sync_copy(rows, acc_sh.at[pl.ds(sid * RPT + k * C, C)])

    plsc.subcore_barrier()

    @pl.loop(0, NCH // GC)
    def _(g):
        slab = wid * NCH + g * GC
        pltpu.sync_copy(src_hbm.at[pl.ds(slab, GC)], idx_s)
        pltpu.sync_copy(dst_hbm.at[pl.ds(slab, GC)], idx_d)

        @pl.loop(0, GC)
        def _(j):
            pltpu.sync_copy(x_hbm.at[idx_s.at[j]], rows)
            pltpu.sync_copy(rows, acc_sh.at[idx_d.at[j]], add=True)

    plsc.subcore_barrier()

    pltpu.sync_copy(acc_sh.at[pl.ds(sid * RPT, RPT)],
                    acc_hbm.at[cid, pl.ds(sid * RPT, RPT)])


def _sc_deg_body(dst_hbm, cnt_hbm, cnt_sh, rows, idx_d):
    cid = lax.axis_index("c")
    sid = lax.axis_index("s")
    wid = cid * NS + sid

    @pl.loop(0, C)
    def _(r):
        @pl.loop(0, D // 16)
        def _(c):
            rows.at[r, pl.ds(c * 16, 16)][...] = jnp.zeros((16,), jnp.float32)

    @pl.loop(0, RPT // C)
    def _(k):
        pltpu.sync_copy(rows, cnt_sh.at[pl.ds(sid * RPT + k * C, C)])

    @pl.loop(0, C)
    def _(r):
        @pl.loop(0, D // 16)
        def _(c):
            rows.at[r, pl.ds(c * 16, 16)][...] = jnp.ones((16,), jnp.float32)

    plsc.subcore_barrier()

    @pl.loop(0, NCH // GC)
    def _(g):
        slab = wid * NCH + g * GC
        pltpu.sync_copy(dst_hbm.at[pl.ds(slab, GC)], idx_d)

        @pl.loop(0, GC)
        def _(j):
            pltpu.sync_copy(rows, cnt_sh.at[idx_d.at[j]], add=True)

    plsc.subcore_barrier()
    pltpu.sync_copy(cnt_sh.at[pl.ds(sid * RPT, RPT)],
                    cnt_hbm.at[cid, pl.ds(sid * RPT, RPT)])


@functools.cache
def _get_sc_agg():
    mesh = plsc.VectorSubcoreMesh(core_axis_name="c", subcore_axis_name="s")
    return pl.kernel(
        _sc_agg_body,
        out_type=[jax.ShapeDtypeStruct((NC, NP, D), jnp.float32)],
        mesh=mesh,
        scratch_types=[
            pltpu.VMEM_SHARED((NP, D), jnp.float32),
            pltpu.VMEM((C, D), jnp.float32),
            pltpu.VMEM((GC, C), jnp.int32),
            pltpu.VMEM((GC, C), jnp.int32),
        ])


@functools.cache
def _get_sc_deg():
    mesh = plsc.VectorSubcoreMesh(core_axis_name="c", subcore_axis_name="s")
    return pl.kernel(
        _sc_deg_body,
        out_type=[jax.ShapeDtypeStruct((NC, NP, D), jnp.float32)],
        mesh=mesh,
        scratch_types=[
            pltpu.VMEM_SHARED((NP, D), jnp.float32),
            pltpu.VMEM((C, D), jnp.float32),
            pltpu.VMEM((GC, C), jnp.int32),
        ])


def _tc_sage_body(acc_ref, cnt_ref, x_ref, wl_ref, wr_ref, b_ref, o_ref):
    acc = acc_ref[0] + acc_ref[1]
    cnt = cnt_ref[0, :, 0:1] + cnt_ref[1, :, 0:1]
    agg = acc / jnp.maximum(cnt, 1.0)
    h = (jnp.dot(agg, wl_ref[...], preferred_element_type=jnp.float32)
         + jnp.dot(x_ref[...], wr_ref[...], preferred_element_type=jnp.float32)
         + b_ref[...])
    o_ref[...] = jnp.maximum(h, 0.0)


def _tc_sage(acc, cnt, x, wl, wr, b):
    return pl.pallas_call(
        _tc_sage_body,
        grid=(NBLK,),
        in_specs=[
            pl.BlockSpec((NC, R, D), lambda i: (0, i, 0)),
            pl.BlockSpec((NC, R, D), lambda i: (0, i, 0)),
            pl.BlockSpec((R, D), lambda i: (i, 0)),
            pl.BlockSpec((D, H), lambda i: (0, 0)),
            pl.BlockSpec((D, H), lambda i: (0, 0)),
            pl.BlockSpec((1, H), lambda i: (0, 0)),
        ],
        out_specs=pl.BlockSpec((R, H), lambda i: (i, 0)),
        out_shape=jax.ShapeDtypeStruct((N, H), jnp.float32),
    )(acc, cnt, x, wl, wr, b)


def _tc_final_body(acc_ref, cnt_ref, h1_ref, batch_ref, wl_ref, wr_ref, b_ref,
                   wc1_ref, bc1_ref, wc2_ref, bc2_ref, o_ref, psum, pcnt):
    i = pl.program_id(0)

    @pl.when(i == 0)
    def _():
        psum[...] = jnp.zeros((G, H), jnp.float32)
        pcnt[...] = jnp.zeros((G, H), jnp.float32)

    acc = acc_ref[0] + acc_ref[1]
    cnt = cnt_ref[0, :, 0:1] + cnt_ref[1, :, 0:1]
    agg = acc / jnp.maximum(cnt, 1.0)
    h2 = (jnp.dot(agg, wl_ref[...], preferred_element_type=jnp.float32)
          + jnp.dot(h1_ref[...], wr_ref[...], preferred_element_type=jnp.float32)
          + b_ref[...])
    h2 = jnp.maximum(h2, 0.0)
    bid = batch_ref[0, 0, :]
    m = (bid[None, :] == lax.broadcasted_iota(jnp.int32, (G, R), 0))
    m = m.astype(jnp.float32)
    psum[...] += jnp.dot(m, h2, preferred_element_type=jnp.float32)
    pcnt[...] += jnp.broadcast_to(jnp.sum(m, axis=1)[:, None], (G, H))

    @pl.when(i == NBLK - 1)
    def _():
        emb = psum[...] / jnp.maximum(pcnt[...], 1.0)
        z = jnp.maximum(
            jnp.dot(emb, wc1_ref[...], preferred_element_type=jnp.float32)
            + bc1_ref[...], 0.0)
        o_ref[0, :] = jnp.sum(z * wc2_ref[...], axis=1) + bc2_ref[0, 0]


def _tc_final(acc, cnt, h1, batch3, wl, wr, b, wc1, bc1, wc2, bc2):
    return pl.pallas_call(
        _tc_final_body,
        grid=(NBLK,),
        in_specs=[
            pl.BlockSpec((NC, R, D), lambda i: (0, i, 0)),
            pl.BlockSpec((NC, R, D), lambda i: (0, i, 0)),
            pl.BlockSpec((R, H), lambda i: (i, 0)),
            pl.BlockSpec((1, 1, R), lambda i: (i, 0, 0)),
            pl.BlockSpec((H, H), lambda i: (0, 0)),
            pl.BlockSpec((H, H), lambda i: (0, 0)),
            pl.BlockSpec((1, H), lambda i: (0, 0)),
            pl.BlockSpec((H, H), lambda i: (0, 0)),
            pl.BlockSpec((1, H), lambda i: (0, 0)),
            pl.BlockSpec((1, H), lambda i: (0, 0)),
            pl.BlockSpec((1, 1), lambda i: (0, 0)),
        ],
        out_specs=pl.BlockSpec((1, G), lambda i: (0, 0)),
        out_shape=jax.ShapeDtypeStruct((1, G), jnp.float32),
        scratch_shapes=[
            pltpu.VMEM((G, H), jnp.float32),
            pltpu.VMEM((G, H), jnp.float32),
        ],
    )(acc, cnt, h1, batch3, wl, wr, b, wc1, bc1, wc2, bc2)


def kernel(x, edge_index, batch, W_l0, b_l0, W_r0, W_l1, b_l1, W_r1,
           W_c1, b_c1, W_c2, b_c2):
    src = edge_index[0]
    dst = edge_index[1]
    pad = EP - E
    srcp = jnp.concatenate(
        [src, jnp.zeros((pad,), src.dtype)]).reshape(NW * NCH, C)
    dstp = jnp.concatenate(
        [dst, jnp.full((pad,), TRASH, dst.dtype)]).reshape(NW * NCH, C)

    cnt, = _get_sc_deg()(dstp)
    acc0, = _get_sc_agg()(x, srcp, dstp)
    h1 = _tc_sage(acc0, cnt, x, W_l0, W_r0, b_l0.reshape(1, H))
    acc1, = _get_sc_agg()(h1, srcp, dstp)
    out = _tc_final(acc1, cnt, h1, batch.reshape(NBLK, 1, R),
                    W_l1, W_r1, b_l1.reshape(1, H),
                    W_c1, b_c1.reshape(1, H), W_c2.reshape(1, H),
                    b_c2.reshape(1, 1))
    return out.reshape(G)


# speedup vs baseline: 7.9589x; 2.5183x over previous
import functools

import jax
import jax.numpy as jnp
from jax import lax
from jax.experimental import pallas as pl
from jax.experimental.pallas import tpu as pltpu
from jax.experimental.pallas import tpu_sc as plsc

N, E, D, H, G = 10000, 320000, 128, 128, 64
NC, NS = 2, 16
NW = NC * NS
C = 128
NCH = 80
GC = 16
EPT = NCH * C
EP = EPT * NW
NP = 10240
RPT = NP // NS
TRASH = N
R = 400
NBLK = N // R


def _sc_agg_body(x_hbm, src_hbm, dst_hbm, acc_hbm, acc_sh, rows0, rows1,
                 idx_s, idx_d, sg0, sg1, ss0, ss1):
    cid = lax.axis_index("c")
    sid = lax.axis_index("s")
    wid = cid * NS + sid

    @pl.loop(0, C)
    def _(r):
        @pl.loop(0, D // 16)
        def _(c):
            rows0.at[r, pl.ds(c * 16, 16)][...] = jnp.zeros((16,), jnp.float32)

    @pl.loop(0, RPT // C)
    def _(k):
        pltpu.sync_copy(rows0, acc_sh.at[pl.ds(sid * RPT + k * C, C)])

    plsc.subcore_barrier()

    @pl.loop(0, NCH // GC)
    def _(g):
        slab = wid * NCH + g * GC
        pltpu.sync_copy(src_hbm.at[pl.ds(slab, GC)], idx_s)
        pltpu.sync_copy(dst_hbm.at[pl.ds(slab, GC)], idx_d)

        @pl.loop(0, GC // 2)
        def _(p):
            c0 = 2 * p
            g0 = pltpu.async_copy(x_hbm.at[idx_s.at[c0]], rows0, sg0)
            g1 = pltpu.async_copy(x_hbm.at[idx_s.at[c0 + 1]], rows1, sg1)
            g0.wait()
            s0 = pltpu.async_copy(rows0, acc_sh.at[idx_d.at[c0]], ss0,
                                  add=True)
            g1.wait()
            s1 = pltpu.async_copy(rows1, acc_sh.at[idx_d.at[c0 + 1]], ss1,
                                  add=True)
            s0.wait()
            s1.wait()

    plsc.subcore_barrier()

    pltpu.sync_copy(acc_sh.at[pl.ds(sid * RPT, RPT)],
                    acc_hbm.at[cid, pl.ds(sid * RPT, RPT)])


def _sc_deg_body(dst_hbm, cnt_hbm, cnt_sh, rows, idx_d, ss0, ss1):
    cid = lax.axis_index("c")
    sid = lax.axis_index("s")
    wid = cid * NS + sid

    @pl.loop(0, C)
    def _(r):
        @pl.loop(0, D // 16)
        def _(c):
            rows.at[r, pl.ds(c * 16, 16)][...] = jnp.zeros((16,), jnp.float32)

    @pl.loop(0, RPT // C)
    def _(k):
        pltpu.sync_copy(rows, cnt_sh.at[pl.ds(sid * RPT + k * C, C)])

    @pl.loop(0, C)
    def _(r):
        @pl.loop(0, D // 16)
        def _(c):
            rows.at[r, pl.ds(c * 16, 16)][...] = jnp.ones((16,), jnp.float32)

    plsc.subcore_barrier()

    @pl.loop(0, NCH // GC)
    def _(g):
        slab = wid * NCH + g * GC
        pltpu.sync_copy(dst_hbm.at[pl.ds(slab, GC)], idx_d)

        @pl.loop(0, GC // 2)
        def _(p):
            c0 = 2 * p
            s0 = pltpu.async_copy(rows, cnt_sh.at[idx_d.at[c0]], ss0,
                                  add=True)
            s1 = pltpu.async_copy(rows, cnt_sh.at[idx_d.at[c0 + 1]], ss1,
                                  add=True)
            s0.wait()
            s1.wait()

    plsc.subcore_barrier()
    pltpu.sync_copy(cnt_sh.at[pl.ds(sid * RPT, RPT)],
                    cnt_hbm.at[cid, pl.ds(sid * RPT, RPT)])


@functools.cache
def _get_sc_agg():
    mesh = plsc.VectorSubcoreMesh(core_axis_name="c", subcore_axis_name="s")
    return pl.kernel(
        _sc_agg_body,
        out_type=[jax.ShapeDtypeStruct((NC, NP, D), jnp.float32)],
        mesh=mesh,
        scratch_types=[
            pltpu.VMEM_SHARED((NP, D), jnp.float32),
            pltpu.VMEM((C, D), jnp.float32),
            pltpu.VMEM((C, D), jnp.float32),
            pltpu.VMEM((GC, C), jnp.int32),
            pltpu.VMEM((GC, C), jnp.int32),
            pltpu.SemaphoreType.DMA,
            pltpu.SemaphoreType.DMA,
            pltpu.SemaphoreType.DMA,
            pltpu.SemaphoreType.DMA,
        ])


@functools.cache
def _get_sc_deg():
    mesh = plsc.VectorSubcoreMesh(core_axis_name="c", subcore_axis_name="s")
    return pl.kernel(
        _sc_deg_body,
        out_type=[jax.ShapeDtypeStruct((NC, NP, D), jnp.float32)],
        mesh=mesh,
        scratch_types=[
            pltpu.VMEM_SHARED((NP, D), jnp.float32),
            pltpu.VMEM((C, D), jnp.float32),
            pltpu.VMEM((GC, C), jnp.int32),
            pltpu.SemaphoreType.DMA,
            pltpu.SemaphoreType.DMA,
        ])


def _tc_sage_body(acc_ref, cnt_ref, x_ref, wl_ref, wr_ref, b_ref, o_ref):
    acc = acc_ref[0] + acc_ref[1]
    cnt = cnt_ref[0, :, 0:1] + cnt_ref[1, :, 0:1]
    agg = acc / jnp.maximum(cnt, 1.0)
    h = (jnp.dot(agg, wl_ref[...], preferred_element_type=jnp.float32)
         + jnp.dot(x_ref[...], wr_ref[...], preferred_element_type=jnp.float32)
         + b_ref[...])
    o_ref[...] = jnp.maximum(h, 0.0)


def _tc_sage(acc, cnt, x, wl, wr, b):
    return pl.pallas_call(
        _tc_sage_body,
        grid=(NBLK,),
        in_specs=[
            pl.BlockSpec((NC, R, D), lambda i: (0, i, 0)),
            pl.BlockSpec((NC, R, D), lambda i: (0, i, 0)),
            pl.BlockSpec((R, D), lambda i: (i, 0)),
            pl.BlockSpec((D, H), lambda i: (0, 0)),
            pl.BlockSpec((D, H), lambda i: (0, 0)),
            pl.BlockSpec((1, H), lambda i: (0, 0)),
        ],
        out_specs=pl.BlockSpec((R, H), lambda i: (i, 0)),
        out_shape=jax.ShapeDtypeStruct((N, H), jnp.float32),
    )(acc, cnt, x, wl, wr, b)


def _tc_final_body(acc_ref, cnt_ref, h1_ref, batch_ref, wl_ref, wr_ref, b_ref,
                   wc1_ref, bc1_ref, wc2_ref, bc2_ref, o_ref, psum, pcnt):
    i = pl.program_id(0)

    @pl.when(i == 0)
    def _():
        psum[...] = jnp.zeros((G, H), jnp.float32)
        pcnt[...] = jnp.zeros((G, H), jnp.float32)

    acc = acc_ref[0] + acc_ref[1]
    cnt = cnt_ref[0, :, 0:1] + cnt_ref[1, :, 0:1]
    agg = acc / jnp.maximum(cnt, 1.0)
    h2 = (jnp.dot(agg, wl_ref[...], preferred_element_type=jnp.float32)
          + jnp.dot(h1_ref[...], wr_ref[...], preferred_element_type=jnp.float32)
          + b_ref[...])
    h2 = jnp.maximum(h2, 0.0)
    bid = batch_ref[0, 0, :]
    m = (bid[None, :] == lax.broadcasted_iota(jnp.int32, (G, R), 0))
    m = m.astype(jnp.float32)
    psum[...] += jnp.dot(m, h2, preferred_element_type=jnp.float32)
    pcnt[...] += jnp.broadcast_to(jnp.sum(m, axis=1)[:, None], (G, H))

    @pl.when(i == NBLK - 1)
    def _():
        emb = psum[...] / jnp.maximum(pcnt[...], 1.0)
        z = jnp.maximum(
            jnp.dot(emb, wc1_ref[...], preferred_element_type=jnp.float32)
            + bc1_ref[...], 0.0)
        o_ref[0, :] = jnp.sum(z * wc2_ref[...], axis=1) + bc2_ref[0, 0]


def _tc_final(acc, cnt, h1, batch3, wl, wr, b, wc1, bc1, wc2, bc2):
    return pl.pallas_call(
        _tc_final_body,
        grid=(NBLK,),
        in_specs=[
            pl.BlockSpec((NC, R, D), lambda i: (0, i, 0)),
            pl.BlockSpec((NC, R, D), lambda i: (0, i, 0)),
            pl.BlockSpec((R, H), lambda i: (i, 0)),
            pl.BlockSpec((1, 1, R), lambda i: (i, 0, 0)),
            pl.BlockSpec((H, H), lambda i: (0, 0)),
            pl.BlockSpec((H, H), lambda i: (0, 0)),
            pl.BlockSpec((1, H), lambda i: (0, 0)),
            pl.BlockSpec((H, H), lambda i: (0, 0)),
            pl.BlockSpec((1, H), lambda i: (0, 0)),
            pl.BlockSpec((1, H), lambda i: (0, 0)),
            pl.BlockSpec((1, 1), lambda i: (0, 0)),
        ],
        out_specs=pl.BlockSpec((1, G), lambda i: (0, 0)),
        out_shape=jax.ShapeDtypeStruct((1, G), jnp.float32),
        scratch_shapes=[
            pltpu.VMEM((G, H), jnp.float32),
            pltpu.VMEM((G, H), jnp.float32),
        ],
    )(acc, cnt, h1, batch3, wl, wr, b, wc1, bc1, wc2, bc2)


def kernel(x, edge_index, batch, W_l0, b_l0, W_r0, W_l1, b_l1, W_r1,
           W_c1, b_c1, W_c2, b_c2):
    src = edge_index[0]
    dst = edge_index[1]
    pad = EP - E
    pad_src = (jnp.arange(pad, dtype=src.dtype) * 37) % N
    pad_dst = TRASH + jnp.arange(pad, dtype=dst.dtype) % (NP - N)
    srcp = jnp.concatenate([src, pad_src]).reshape(NW * NCH, C)
    dstp = jnp.concatenate([dst, pad_dst]).reshape(NW * NCH, C)

    cnt, = _get_sc_deg()(dstp)
    acc0, = _get_sc_agg()(x, srcp, dstp)
    h1 = _tc_sage(acc0, cnt, x, W_l0, W_r0, b_l0.reshape(1, H))
    acc1, = _get_sc_agg()(h1, srcp, dstp)
    out = _tc_final(acc1, cnt, h1, batch.reshape(NBLK, 1, R),
                    W_l1, W_r1, b_l1.reshape(1, H),
                    W_c1, b_c1.reshape(1, H), W_c2.reshape(1, H),
                    b_c2.reshape(1, 1))
    return out.reshape(G)


# speedup vs baseline: 8.2583x; 1.0376x over previous
import functools

import jax
import jax.numpy as jnp
from jax import lax
from jax.experimental import pallas as pl
from jax.experimental.pallas import tpu as pltpu
from jax.experimental.pallas import tpu_sc as plsc

N, E, D, H, G = 10000, 320000, 128, 128, 64
NC, NS = 2, 16
NW = NC * NS
C = 128
NCH = 80
GC = 16
EPT = NCH * C
EP = EPT * NW
NP = 10240
RPT = NP // NS
TRASH = N
R = 400
NBLK = N // R


def _sc_agg_body(x_hbm, src_hbm, dst_hbm, acc_hbm, acc_sh, rows0, rows1,
                 idx_s0, idx_d0, idx_s1, idx_d1, sg0, sg1, ss0, ss1, si):
    cid = lax.axis_index("c")
    sid = lax.axis_index("s")
    wid = cid * NS + sid

    @pl.loop(0, C)
    def _(r):
        @pl.loop(0, D // 16)
        def _(c):
            rows0.at[r, pl.ds(c * 16, 16)][...] = jnp.zeros((16,), jnp.float32)

    @pl.loop(0, RPT // C)
    def _(k):
        pltpu.sync_copy(rows0, acc_sh.at[pl.ds(sid * RPT + k * C, C)])

    plsc.subcore_barrier()

    def _wait_gather(buf, sem):
        pltpu.make_async_copy(x_hbm.at[pl.ds(0, C)], buf, sem).wait()

    def _wait_scatter(buf, sem):
        pltpu.make_async_copy(buf, acc_sh.at[pl.ds(0, C)], sem).wait()

    pltpu.sync_copy(src_hbm.at[pl.ds(wid * NCH, GC)], idx_s0)
    pltpu.sync_copy(dst_hbm.at[pl.ds(wid * NCH, GC)], idx_d0)
    pltpu.async_copy(x_hbm.at[idx_s0.at[0]], rows0, sg0)

    ng = NCH // GC
    for g in range(ng):
        isb, idb = (idx_s0, idx_d0) if g % 2 == 0 else (idx_s1, idx_d1)
        isn, idn = (idx_s1, idx_d1) if g % 2 == 0 else (idx_s0, idx_d0)
        if g + 1 < ng:
            nslab = wid * NCH + (g + 1) * GC
            pltpu.async_copy(src_hbm.at[pl.ds(nslab, GC)], isn, si)
            pltpu.async_copy(dst_hbm.at[pl.ds(nslab, GC)], idn, si)

        @pl.loop(0, GC // 2)
        def _(p):
            c0 = 2 * p
            pltpu.async_copy(x_hbm.at[isb.at[c0 + 1]], rows1, sg1)
            _wait_gather(rows0, sg0)
            pltpu.async_copy(rows0, acc_sh.at[idb.at[c0]], ss0, add=True)
            _wait_gather(rows1, sg1)
            pltpu.async_copy(rows1, acc_sh.at[idb.at[c0 + 1]], ss1, add=True)
            _wait_scatter(rows0, ss0)

            @pl.when(p < GC // 2 - 1)
            def _():
                pltpu.async_copy(x_hbm.at[isb.at[c0 + 2]], rows0, sg0)

            _wait_scatter(rows1, ss1)

        if g + 1 < ng:
            pltpu.make_async_copy(src_hbm.at[pl.ds(wid * NCH, GC)], isn,
                                  si).wait()
            pltpu.make_async_copy(dst_hbm.at[pl.ds(wid * NCH, GC)], idn,
                                  si).wait()
            pltpu.async_copy(x_hbm.at[isn.at[0]], rows0, sg0)

    plsc.subcore_barrier()

    pltpu.sync_copy(acc_sh.at[pl.ds(sid * RPT, RPT)],
                    acc_hbm.at[cid, pl.ds(sid * RPT, RPT)])


def _sc_deg_body(dst_hbm, cnt_hbm, cnt_sh, rows, idx_d, ss0, ss1):
    cid = lax.axis_index("c")
    sid = lax.axis_index("s")
    wid = cid * NS + sid

    @pl.loop(0, C)
    def _(r):
        @pl.loop(0, D // 16)
        def _(c):
            rows.at[r, pl.ds(c * 16, 16)][...] = jnp.zeros((16,), jnp.float32)

    @pl.loop(0, RPT // C)
    def _(k):
        pltpu.sync_copy(rows, cnt_sh.at[pl.ds(sid * RPT + k * C, C)])

    @pl.loop(0, C)
    def _(r):
        @pl.loop(0, D // 16)
        def _(c):
            rows.at[r, pl.ds(c * 16, 16)][...] = jnp.ones((16,), jnp.float32)

    plsc.subcore_barrier()

    @pl.loop(0, NCH // GC)
    def _(g):
        slab = wid * NCH + g * GC
        pltpu.sync_copy(dst_hbm.at[pl.ds(slab, GC)], idx_d)

        @pl.loop(0, GC // 2)
        def _(p):
            c0 = 2 * p
            s0 = pltpu.async_copy(rows, cnt_sh.at[idx_d.at[c0]], ss0,
                                  add=True)
            s1 = pltpu.async_copy(rows, cnt_sh.at[idx_d.at[c0 + 1]], ss1,
                                  add=True)
            s0.wait()
            s1.wait()

    plsc.subcore_barrier()
    pltpu.sync_copy(cnt_sh.at[pl.ds(sid * RPT, RPT)],
                    cnt_hbm.at[cid, pl.ds(sid * RPT, RPT)])


@functools.cache
def _get_sc_agg():
    mesh = plsc.VectorSubcoreMesh(core_axis_name="c", subcore_axis_name="s")
    return pl.kernel(
        _sc_agg_body,
        out_type=[jax.ShapeDtypeStruct((NC, NP, D), jnp.float32)],
        mesh=mesh,
        scratch_types=[
            pltpu.VMEM_SHARED((NP, D), jnp.float32),
            pltpu.VMEM((C, D), jnp.float32),
            pltpu.VMEM((C, D), jnp.float32),
            pltpu.VMEM((GC, C), jnp.int32),
            pltpu.VMEM((GC, C), jnp.int32),
            pltpu.VMEM((GC, C), jnp.int32),
            pltpu.VMEM((GC, C), jnp.int32),
            pltpu.SemaphoreType.DMA,
            pltpu.SemaphoreType.DMA,
            pltpu.SemaphoreType.DMA,
            pltpu.SemaphoreType.DMA,
            pltpu.SemaphoreType.DMA,
        ])


@functools.cache
def _get_sc_deg():
    mesh = plsc.VectorSubcoreMesh(core_axis_name="c", subcore_axis_name="s")
    return pl.kernel(
        _sc_deg_body,
        out_type=[jax.ShapeDtypeStruct((NC, NP, D), jnp.float32)],
        mesh=mesh,
        scratch_types=[
            pltpu.VMEM_SHARED((NP, D), jnp.float32),
            pltpu.VMEM((C, D), jnp.float32),
            pltpu.VMEM((GC, C), jnp.int32),
            pltpu.SemaphoreType.DMA,
            pltpu.SemaphoreType.DMA,
        ])


def _tc_sage_body(acc_ref, cnt_ref, x_ref, wl_ref, wr_ref, b_ref, o_ref):
    acc = acc_ref[0] + acc_ref[1]
    cnt = cnt_ref[0, :, 0:1] + cnt_ref[1, :, 0:1]
    agg = acc / jnp.maximum(cnt, 1.0)
    h = (jnp.dot(agg, wl_ref[...], preferred_element_type=jnp.float32)
         + jnp.dot(x_ref[...], wr_ref[...], preferred_element_type=jnp.float32)
         + b_ref[...])
    o_ref[...] = jnp.maximum(h, 0.0)


def _tc_sage(acc, cnt, x, wl, wr, b):
    return pl.pallas_call(
        _tc_sage_body,
        grid=(NBLK,),
        in_specs=[
            pl.BlockSpec((NC, R, D), lambda i: (0, i, 0)),
            pl.BlockSpec((NC, R, D), lambda i: (0, i, 0)),
            pl.BlockSpec((R, D), lambda i: (i, 0)),
            pl.BlockSpec((D, H), lambda i: (0, 0)),
            pl.BlockSpec((D, H), lambda i: (0, 0)),
            pl.BlockSpec((1, H), lambda i: (0, 0)),
        ],
        out_specs=pl.BlockSpec((R, H), lambda i: (i, 0)),
        out_shape=jax.ShapeDtypeStruct((N, H), jnp.float32),
    )(acc, cnt, x, wl, wr, b)


def _tc_final_body(acc_ref, cnt_ref, h1_ref, batch_ref, wl_ref, wr_ref, b_ref,
                   wc1_ref, bc1_ref, wc2_ref, bc2_ref, o_ref, psum, pcnt):
    i = pl.program_id(0)

    @pl.when(i == 0)
    def _():
        psum[...] = jnp.zeros((G, H), jnp.float32)
        pcnt[...] = jnp.zeros((G, H), jnp.float32)

    acc = acc_ref[0] + acc_ref[1]
    cnt = cnt_ref[0, :, 0:1] + cnt_ref[1, :, 0:1]
    agg = acc / jnp.maximum(cnt, 1.0)
    h2 = (jnp.dot(agg, wl_ref[...], preferred_element_type=jnp.float32)
          + jnp.dot(h1_ref[...], wr_ref[...], preferred_element_type=jnp.float32)
          + b_ref[...])
    h2 = jnp.maximum(h2, 0.0)
    bid = batch_ref[0, 0, :]
    m = (bid[None, :] == lax.broadcasted_iota(jnp.int32, (G, R), 0))
    m = m.astype(jnp.float32)
    psum[...] += jnp.dot(m, h2, preferred_element_type=jnp.float32)
    pcnt[...] += jnp.broadcast_to(jnp.sum(m, axis=1)[:, None], (G, H))

    @pl.when(i == NBLK - 1)
    def _():
        emb = psum[...] / jnp.maximum(pcnt[...], 1.0)
        z = jnp.maximum(
            jnp.dot(emb, wc1_ref[...], preferred_element_type=jnp.float32)
            + bc1_ref[...], 0.0)
        o_ref[0, :] = jnp.sum(z * wc2_ref[...], axis=1) + bc2_ref[0, 0]


def _tc_final(acc, cnt, h1, batch3, wl, wr, b, wc1, bc1, wc2, bc2):
    return pl.pallas_call(
        _tc_final_body,
        grid=(NBLK,),
        in_specs=[
            pl.BlockSpec((NC, R, D), lambda i: (0, i, 0)),
            pl.BlockSpec((NC, R, D), lambda i: (0, i, 0)),
            pl.BlockSpec((R, H), lambda i: (i, 0)),
            pl.BlockSpec((1, 1, R), lambda i: (i, 0, 0)),
            pl.BlockSpec((H, H), lambda i: (0, 0)),
            pl.BlockSpec((H, H), lambda i: (0, 0)),
            pl.BlockSpec((1, H), lambda i: (0, 0)),
            pl.BlockSpec((H, H), lambda i: (0, 0)),
            pl.BlockSpec((1, H), lambda i: (0, 0)),
            pl.BlockSpec((1, H), lambda i: (0, 0)),
            pl.BlockSpec((1, 1), lambda i: (0, 0)),
        ],
        out_specs=pl.BlockSpec((1, G), lambda i: (0, 0)),
        out_shape=jax.ShapeDtypeStruct((1, G), jnp.float32),
        scratch_shapes=[
            pltpu.VMEM((G, H), jnp.float32),
            pltpu.VMEM((G, H), jnp.float32),
        ],
    )(acc, cnt, h1, batch3, wl, wr, b, wc1, bc1, wc2, bc2)


def kernel(x, edge_index, batch, W_l0, b_l0, W_r0, W_l1, b_l1, W_r1,
           W_c1, b_c1, W_c2, b_c2):
    src = edge_index[0]
    dst = edge_index[1]
    pad = EP - E
    pad_src = (jnp.arange(pad, dtype=src.dtype) * 37) % N
    pad_dst = TRASH + jnp.arange(pad, dtype=dst.dtype) % (NP - N)
    srcp = jnp.concatenate([src, pad_src]).reshape(NW * NCH, C)
    dstp = jnp.concatenate([dst, pad_dst]).reshape(NW * NCH, C)

    cnt, = _get_sc_deg()(dstp)
    acc0, = _get_sc_agg()(x, srcp, dstp)
    h1 = _tc_sage(acc0, cnt, x, W_l0, W_r0, b_l0.reshape(1, H))
    acc1, = _get_sc_agg()(h1, srcp, dstp)
    out = _tc_final(acc1, cnt, h1, batch.reshape(NBLK, 1, R),
                    W_l1, W_r1, b_l1.reshape(1, H),
                    W_c1, b_c1.reshape(1, H), W_c2.reshape(1, H),
                    b_c2.reshape(1, 1))
    return out.reshape(G)


# speedup vs baseline: 8.5773x; 1.0386x over previous
import functools

import jax
import jax.numpy as jnp
from jax import lax
from jax.experimental import pallas as pl
from jax.experimental.pallas import tpu as pltpu
from jax.experimental.pallas import tpu_sc as plsc

N, E, D, H, G = 10000, 320000, 128, 128, 64
NC, NS = 2, 16
NW = NC * NS
C = 128
NCH = 80
GC = 16
EPT = NCH * C
EP = EPT * NW
NP = 10240
RPT = NP // NS
TRASH = N
R = 1000
NBLK = N // R


def _sc_agg_body(x_hbm, src_hbm, dst_hbm, acc_hbm, acc_sh, rows0, rows1,
                 idx_s0, idx_d0, idx_s1, idx_d1, sg0, sg1, ss0, ss1, si):
    cid = lax.axis_index("c")
    sid = lax.axis_index("s")
    wid = cid * NS + sid

    @pl.loop(0, C)
    def _(r):
        @pl.loop(0, D // 16)
        def _(c):
            rows0.at[r, pl.ds(c * 16, 16)][...] = jnp.zeros((16,), jnp.float32)

    @pl.loop(0, RPT // C)
    def _(k):
        pltpu.sync_copy(rows0, acc_sh.at[pl.ds(sid * RPT + k * C, C)])

    plsc.subcore_barrier()

    def _wait_gather(buf, sem):
        pltpu.make_async_copy(x_hbm.at[pl.ds(0, C)], buf, sem).wait()

    def _wait_scatter(buf, sem):
        pltpu.make_async_copy(buf, acc_sh.at[pl.ds(0, C)], sem).wait()

    def _fire_gather(islab, c, buf, sem):
        pltpu.async_copy(x_hbm.at[islab.at[c, pl.ds(0, C // 2)]],
                         buf.at[pl.ds(0, C // 2)], sem)
        pltpu.async_copy(x_hbm.at[islab.at[c, pl.ds(C // 2, C // 2)]],
                         buf.at[pl.ds(C // 2, C // 2)], sem)

    pltpu.sync_copy(src_hbm.at[pl.ds(wid * NCH, GC)], idx_s0)
    pltpu.sync_copy(dst_hbm.at[pl.ds(wid * NCH, GC)], idx_d0)
    _fire_gather(idx_s0, 0, rows0, sg0)

    ng = NCH // GC
    for g in range(ng):
        isb, idb = (idx_s0, idx_d0) if g % 2 == 0 else (idx_s1, idx_d1)
        isn, idn = (idx_s1, idx_d1) if g % 2 == 0 else (idx_s0, idx_d0)
        if g + 1 < ng:
            nslab = wid * NCH + (g + 1) * GC
            pltpu.async_copy(src_hbm.at[pl.ds(nslab, GC)], isn, si)
            pltpu.async_copy(dst_hbm.at[pl.ds(nslab, GC)], idn, si)

        @pl.loop(0, GC // 2)
        def _(p):
            c0 = 2 * p
            _fire_gather(isb, c0 + 1, rows1, sg1)
            _wait_gather(rows0, sg0)
            pltpu.async_copy(rows0, acc_sh.at[idb.at[c0]], ss0, add=True)
            _wait_gather(rows1, sg1)
            pltpu.async_copy(rows1, acc_sh.at[idb.at[c0 + 1]], ss1, add=True)
            _wait_scatter(rows0, ss0)

            @pl.when(p < GC // 2 - 1)
            def _():
                _fire_gather(isb, c0 + 2, rows0, sg0)

            _wait_scatter(rows1, ss1)

        if g + 1 < ng:
            pltpu.make_async_copy(src_hbm.at[pl.ds(wid * NCH, GC)], isn,
                                  si).wait()
            pltpu.make_async_copy(dst_hbm.at[pl.ds(wid * NCH, GC)], idn,
                                  si).wait()
            _fire_gather(isn, 0, rows0, sg0)

    plsc.subcore_barrier()

    pltpu.sync_copy(acc_sh.at[pl.ds(sid * RPT, RPT)],
                    acc_hbm.at[cid, pl.ds(sid * RPT, RPT)])


def _sc_deg_body(dst_hbm, cnt_hbm, cnt_sh, rows, idx_d, ss0, ss1):
    cid = lax.axis_index("c")
    sid = lax.axis_index("s")
    wid = cid * NS + sid

    @pl.loop(0, C)
    def _(r):
        @pl.loop(0, D // 16)
        def _(c):
            rows.at[r, pl.ds(c * 16, 16)][...] = jnp.zeros((16,), jnp.float32)

    @pl.loop(0, RPT // C)
    def _(k):
        pltpu.sync_copy(rows, cnt_sh.at[pl.ds(sid * RPT + k * C, C)])

    @pl.loop(0, C)
    def _(r):
        @pl.loop(0, D // 16)
        def _(c):
            rows.at[r, pl.ds(c * 16, 16)][...] = jnp.ones((16,), jnp.float32)

    plsc.subcore_barrier()

    @pl.loop(0, NCH // GC)
    def _(g):
        slab = wid * NCH + g * GC
        pltpu.sync_copy(dst_hbm.at[pl.ds(slab, GC)], idx_d)

        @pl.loop(0, GC // 2)
        def _(p):
            c0 = 2 * p
            s0 = pltpu.async_copy(rows, cnt_sh.at[idx_d.at[c0]], ss0,
                                  add=True)
            s1 = pltpu.async_copy(rows, cnt_sh.at[idx_d.at[c0 + 1]], ss1,
                                  add=True)
            s0.wait()
            s1.wait()

    plsc.subcore_barrier()
    pltpu.sync_copy(cnt_sh.at[pl.ds(sid * RPT, RPT)],
                    cnt_hbm.at[cid, pl.ds(sid * RPT, RPT)])


@functools.cache
def _get_sc_agg():
    mesh = plsc.VectorSubcoreMesh(core_axis_name="c", subcore_axis_name="s")
    return pl.kernel(
        _sc_agg_body,
        out_type=[jax.ShapeDtypeStruct((NC, NP, D), jnp.float32)],
        mesh=mesh,
        scratch_types=[
            pltpu.VMEM_SHARED((NP, D), jnp.float32),
            pltpu.VMEM((C, D), jnp.float32),
            pltpu.VMEM((C, D), jnp.float32),
            pltpu.VMEM((GC, C), jnp.int32),
            pltpu.VMEM((GC, C), jnp.int32),
            pltpu.VMEM((GC, C), jnp.int32),
            pltpu.VMEM((GC, C), jnp.int32),
            pltpu.SemaphoreType.DMA,
            pltpu.SemaphoreType.DMA,
            pltpu.SemaphoreType.DMA,
            pltpu.SemaphoreType.DMA,
            pltpu.SemaphoreType.DMA,
        ])


@functools.cache
def _get_sc_deg():
    mesh = plsc.VectorSubcoreMesh(core_axis_name="c", subcore_axis_name="s")
    return pl.kernel(
        _sc_deg_body,
        out_type=[jax.ShapeDtypeStruct((NC, NP, D), jnp.float32)],
        mesh=mesh,
        scratch_types=[
            pltpu.VMEM_SHARED((NP, D), jnp.float32),
            pltpu.VMEM((C, D), jnp.float32),
            pltpu.VMEM((GC, C), jnp.int32),
            pltpu.SemaphoreType.DMA,
            pltpu.SemaphoreType.DMA,
        ])


def _tc_sage_body(acc_ref, cnt_ref, x_ref, wl_ref, wr_ref, b_ref, o_ref):
    acc = acc_ref[0] + acc_ref[1]
    cnt = cnt_ref[0, :, 0:1] + cnt_ref[1, :, 0:1]
    agg = acc / jnp.maximum(cnt, 1.0)
    h = (jnp.dot(agg, wl_ref[...], preferred_element_type=jnp.float32)
         + jnp.dot(x_ref[...], wr_ref[...], preferred_element_type=jnp.float32)
         + b_ref[...])
    o_ref[...] = jnp.maximum(h, 0.0)


def _tc_sage(acc, cnt, x, wl, wr, b):
    return pl.pallas_call(
        _tc_sage_body,
        grid=(NBLK,),
        in_specs=[
            pl.BlockSpec((NC, R, D), lambda i: (0, i, 0)),
            pl.BlockSpec((NC, R, D), lambda i: (0, i, 0)),
            pl.BlockSpec((R, D), lambda i: (i, 0)),
            pl.BlockSpec((D, H), lambda i: (0, 0)),
            pl.BlockSpec((D, H), lambda i: (0, 0)),
            pl.BlockSpec((1, H), lambda i: (0, 0)),
        ],
        out_specs=pl.BlockSpec((R, H), lambda i: (i, 0)),
        out_shape=jax.ShapeDtypeStruct((N, H), jnp.float32),
    )(acc, cnt, x, wl, wr, b)


def _tc_final_body(acc_ref, cnt_ref, h1_ref, batch_ref, wl_ref, wr_ref, b_ref,
                   wc1_ref, bc1_ref, wc2_ref, bc2_ref, o_ref, psum, pcnt):
    i = pl.program_id(0)

    @pl.when(i == 0)
    def _():
        psum[...] = jnp.zeros((G, H), jnp.float32)
        pcnt[...] = jnp.zeros((G, H), jnp.float32)

    acc = acc_ref[0] + acc_ref[1]
    cnt = cnt_ref[0, :, 0:1] + cnt_ref[1, :, 0:1]
    agg = acc / jnp.maximum(cnt, 1.0)
    h2 = (jnp.dot(agg, wl_ref[...], preferred_element_type=jnp.float32)
          + jnp.dot(h1_ref[...], wr_ref[...], preferred_element_type=jnp.float32)
          + b_ref[...])
    h2 = jnp.maximum(h2, 0.0)
    bid = batch_ref[0, 0, :]
    m = (bid[None, :] == lax.broadcasted_iota(jnp.int32, (G, R), 0))
    m = m.astype(jnp.float32)
    psum[...] += jnp.dot(m, h2, preferred_element_type=jnp.float32)
    pcnt[...] += jnp.broadcast_to(jnp.sum(m, axis=1)[:, None], (G, H))

    @pl.when(i == NBLK - 1)
    def _():
        emb = psum[...] / jnp.maximum(pcnt[...], 1.0)
        z = jnp.maximum(
            jnp.dot(emb, wc1_ref[...], preferred_element_type=jnp.float32)
            + bc1_ref[...], 0.0)
        o_ref[0, :] = jnp.sum(z * wc2_ref[...], axis=1) + bc2_ref[0, 0]


def _tc_final(acc, cnt, h1, batch3, wl, wr, b, wc1, bc1, wc2, bc2):
    return pl.pallas_call(
        _tc_final_body,
        grid=(NBLK,),
        in_specs=[
            pl.BlockSpec((NC, R, D), lambda i: (0, i, 0)),
            pl.BlockSpec((NC, R, D), lambda i: (0, i, 0)),
            pl.BlockSpec((R, H), lambda i: (i, 0)),
            pl.BlockSpec((1, 1, R), lambda i: (i, 0, 0)),
            pl.BlockSpec((H, H), lambda i: (0, 0)),
            pl.BlockSpec((H, H), lambda i: (0, 0)),
            pl.BlockSpec((1, H), lambda i: (0, 0)),
            pl.BlockSpec((H, H), lambda i: (0, 0)),
            pl.BlockSpec((1, H), lambda i: (0, 0)),
            pl.BlockSpec((1, H), lambda i: (0, 0)),
            pl.BlockSpec((1, 1), lambda i: (0, 0)),
        ],
        out_specs=pl.BlockSpec((1, G), lambda i: (0, 0)),
        out_shape=jax.ShapeDtypeStruct((1, G), jnp.float32),
        scratch_shapes=[
            pltpu.VMEM((G, H), jnp.float32),
            pltpu.VMEM((G, H), jnp.float32),
        ],
    )(acc, cnt, h1, batch3, wl, wr, b, wc1, bc1, wc2, bc2)


def kernel(x, edge_index, batch, W_l0, b_l0, W_r0, W_l1, b_l1, W_r1,
           W_c1, b_c1, W_c2, b_c2):
    src = edge_index[0]
    dst = edge_index[1]
    pad = EP - E
    pad_src = (jnp.arange(pad, dtype=src.dtype) * 37) % N
    pad_dst = TRASH + jnp.arange(pad, dtype=dst.dtype) % (NP - N)
    srcp = jnp.concatenate([src, pad_src]).reshape(NW * NCH, C)
    dstp = jnp.concatenate([dst, pad_dst]).reshape(NW * NCH, C)

    cnt, = _get_sc_deg()(dstp)
    acc0, = _get_sc_agg()(x, srcp, dstp)
    h1 = _tc_sage(acc0, cnt, x, W_l0, W_r0, b_l0.reshape(1, H))
    acc1, = _get_sc_agg()(h1, srcp, dstp)
    out = _tc_final(acc1, cnt, h1, batch.reshape(NBLK, 1, R),
                    W_l1, W_r1, b_l1.reshape(1, H),
                    W_c1, b_c1.reshape(1, H), W_c2.reshape(1, H),
                    b_c2.reshape(1, 1))
    return out.reshape(G)


# speedup vs baseline: 8.7106x; 1.0155x over previous
import functools

import jax
import jax.numpy as jnp
from jax import lax
from jax.experimental import pallas as pl
from jax.experimental.pallas import tpu as pltpu
from jax.experimental.pallas import tpu_sc as plsc

N, E, D, H, G = 10000, 320000, 128, 128, 64
NC, NS = 2, 16
NW = NC * NS
C = 128
NCH = 80
GC = 16
EPT = NCH * C
EP = EPT * NW
NP = 10240
RPT = NP // NS
TRASH = N
R = 1000
NBLK = N // R


def _sc_agg_body(x_hbm, edges_hbm, acc_hbm, acc_sh, rows0, rows1,
                 idx_s0, idx_d0, idx_s1, idx_d1, sg0, sg1, ss0, ss1, si):
    src_hbm = edges_hbm.at[0]
    dst_hbm = edges_hbm.at[1]
    cid = lax.axis_index("c")
    sid = lax.axis_index("s")
    wid = cid * NS + sid

    @pl.loop(0, C)
    def _(r):
        @pl.loop(0, D // 16)
        def _(c):
            rows0.at[r, pl.ds(c * 16, 16)][...] = jnp.zeros((16,), jnp.float32)

    @pl.loop(0, RPT // C)
    def _(k):
        pltpu.sync_copy(rows0, acc_sh.at[pl.ds(sid * RPT + k * C, C)])

    plsc.subcore_barrier()

    def _wait_gather(buf, sem):
        pltpu.make_async_copy(x_hbm.at[pl.ds(0, C)], buf, sem).wait()

    def _wait_scatter(buf, sem):
        pltpu.make_async_copy(buf, acc_sh.at[pl.ds(0, C)], sem).wait()

    def _fire_gather(islab, c, buf, sem):
        pltpu.async_copy(x_hbm.at[islab.at[c, pl.ds(0, C // 2)]],
                         buf.at[pl.ds(0, C // 2)], sem)
        pltpu.async_copy(x_hbm.at[islab.at[c, pl.ds(C // 2, C // 2)]],
                         buf.at[pl.ds(C // 2, C // 2)], sem)

    pltpu.sync_copy(src_hbm.at[pl.ds(wid * NCH, GC)], idx_s0)
    pltpu.sync_copy(dst_hbm.at[pl.ds(wid * NCH, GC)], idx_d0)
    _fire_gather(idx_s0, 0, rows0, sg0)

    ng = NCH // GC
    for g in range(ng):
        isb, idb = (idx_s0, idx_d0) if g % 2 == 0 else (idx_s1, idx_d1)
        isn, idn = (idx_s1, idx_d1) if g % 2 == 0 else (idx_s0, idx_d0)
        if g + 1 < ng:
            nslab = wid * NCH + (g + 1) * GC
            pltpu.async_copy(src_hbm.at[pl.ds(nslab, GC)], isn, si)
            pltpu.async_copy(dst_hbm.at[pl.ds(nslab, GC)], idn, si)

        @pl.loop(0, GC // 2)
        def _(p):
            c0 = 2 * p
            _fire_gather(isb, c0 + 1, rows1, sg1)
            _wait_gather(rows0, sg0)
            pltpu.async_copy(rows0, acc_sh.at[idb.at[c0]], ss0, add=True)
            _wait_gather(rows1, sg1)
            pltpu.async_copy(rows1, acc_sh.at[idb.at[c0 + 1]], ss1, add=True)
            _wait_scatter(rows0, ss0)

            @pl.when(p < GC // 2 - 1)
            def _():
                _fire_gather(isb, c0 + 2, rows0, sg0)

            _wait_scatter(rows1, ss1)

        if g + 1 < ng:
            pltpu.make_async_copy(src_hbm.at[pl.ds(wid * NCH, GC)], isn,
                                  si).wait()
            pltpu.make_async_copy(dst_hbm.at[pl.ds(wid * NCH, GC)], idn,
                                  si).wait()
            _fire_gather(isn, 0, rows0, sg0)

    plsc.subcore_barrier()

    pltpu.sync_copy(acc_sh.at[pl.ds(sid * RPT, RPT)],
                    acc_hbm.at[cid, pl.ds(sid * RPT, RPT)])


def _sc_deg_body(edges_hbm, cnt_hbm, cnt_sh, rows, idx_d, ss0, ss1):
    dst_hbm = edges_hbm.at[1]
    cid = lax.axis_index("c")
    sid = lax.axis_index("s")
    wid = cid * NS + sid

    @pl.loop(0, C)
    def _(r):
        @pl.loop(0, D // 16)
        def _(c):
            rows.at[r, pl.ds(c * 16, 16)][...] = jnp.zeros((16,), jnp.float32)

    @pl.loop(0, RPT // C)
    def _(k):
        pltpu.sync_copy(rows, cnt_sh.at[pl.ds(sid * RPT + k * C, C)])

    @pl.loop(0, C)
    def _(r):
        @pl.loop(0, D // 16)
        def _(c):
            rows.at[r, pl.ds(c * 16, 16)][...] = jnp.ones((16,), jnp.float32)

    plsc.subcore_barrier()

    @pl.loop(0, NCH // GC)
    def _(g):
        slab = wid * NCH + g * GC
        pltpu.sync_copy(dst_hbm.at[pl.ds(slab, GC)], idx_d)

        @pl.loop(0, GC // 2)
        def _(p):
            c0 = 2 * p
            s0 = pltpu.async_copy(rows, cnt_sh.at[idx_d.at[c0]], ss0,
                                  add=True)
            s1 = pltpu.async_copy(rows, cnt_sh.at[idx_d.at[c0 + 1]], ss1,
                                  add=True)
            s0.wait()
            s1.wait()

    plsc.subcore_barrier()
    pltpu.sync_copy(cnt_sh.at[pl.ds(sid * RPT, RPT)],
                    cnt_hbm.at[cid, pl.ds(sid * RPT, RPT)])


@functools.cache
def _get_sc_agg():
    mesh = plsc.VectorSubcoreMesh(core_axis_name="c", subcore_axis_name="s")
    return pl.kernel(
        _sc_agg_body,
        out_type=[jax.ShapeDtypeStruct((NC, NP, D), jnp.float32)],
        mesh=mesh,
        scratch_types=[
            pltpu.VMEM_SHARED((NP, D), jnp.float32),
            pltpu.VMEM((C, D), jnp.float32),
            pltpu.VMEM((C, D), jnp.float32),
            pltpu.VMEM((GC, C), jnp.int32),
            pltpu.VMEM((GC, C), jnp.int32),
            pltpu.VMEM((GC, C), jnp.int32),
            pltpu.VMEM((GC, C), jnp.int32),
            pltpu.SemaphoreType.DMA,
            pltpu.SemaphoreType.DMA,
            pltpu.SemaphoreType.DMA,
            pltpu.SemaphoreType.DMA,
            pltpu.SemaphoreType.DMA,
        ])


@functools.cache
def _get_sc_deg():
    mesh = plsc.VectorSubcoreMesh(core_axis_name="c", subcore_axis_name="s")
    return pl.kernel(
        _sc_deg_body,
        out_type=[jax.ShapeDtypeStruct((NC, NP, D), jnp.float32)],
        mesh=mesh,
        scratch_types=[
            pltpu.VMEM_SHARED((NP, D), jnp.float32),
            pltpu.VMEM((C, D), jnp.float32),
            pltpu.VMEM((GC, C), jnp.int32),
            pltpu.SemaphoreType.DMA,
            pltpu.SemaphoreType.DMA,
        ])


def _tc_sage_body(acc_ref, cnt_ref, x_ref, wl_ref, wr_ref, b_ref, o_ref):
    acc = acc_ref[0] + acc_ref[1]
    cnt = cnt_ref[0, :, 0:1] + cnt_ref[1, :, 0:1]
    agg = acc / jnp.maximum(cnt, 1.0)
    h = (jnp.dot(agg, wl_ref[...], preferred_element_type=jnp.float32)
         + jnp.dot(x_ref[...], wr_ref[...], preferred_element_type=jnp.float32)
         + b_ref[...])
    o_ref[...] = jnp.maximum(h, 0.0)


def _tc_sage(acc, cnt, x, wl, wr, b):
    return pl.pallas_call(
        _tc_sage_body,
        grid=(NBLK,),
        in_specs=[
            pl.BlockSpec((NC, R, D), lambda i: (0, i, 0)),
            pl.BlockSpec((NC, R, D), lambda i: (0, i, 0)),
            pl.BlockSpec((R, D), lambda i: (i, 0)),
            pl.BlockSpec((D, H), lambda i: (0, 0)),
            pl.BlockSpec((D, H), lambda i: (0, 0)),
            pl.BlockSpec((1, H), lambda i: (0, 0)),
        ],
        out_specs=pl.BlockSpec((R, H), lambda i: (i, 0)),
        out_shape=jax.ShapeDtypeStruct((N, H), jnp.float32),
    )(acc, cnt, x, wl, wr, b)


def _tc_final_body(acc_ref, cnt_ref, h1_ref, batch_ref, wl_ref, wr_ref, b_ref,
                   wc1_ref, bc1_ref, wc2_ref, bc2_ref, o_ref, psum, pcnt):
    i = pl.program_id(0)

    @pl.when(i == 0)
    def _():
        psum[...] = jnp.zeros((G, H), jnp.float32)
        pcnt[...] = jnp.zeros((G, H), jnp.float32)

    acc = acc_ref[0] + acc_ref[1]
    cnt = cnt_ref[0, :, 0:1] + cnt_ref[1, :, 0:1]
    agg = acc / jnp.maximum(cnt, 1.0)
    h2 = (jnp.dot(agg, wl_ref[...], preferred_element_type=jnp.float32)
          + jnp.dot(h1_ref[...], wr_ref[...], preferred_element_type=jnp.float32)
          + b_ref[...])
    h2 = jnp.maximum(h2, 0.0)
    bid = batch_ref[0, 0, :]
    m = (bid[None, :] == lax.broadcasted_iota(jnp.int32, (G, R), 0))
    m = m.astype(jnp.float32)
    psum[...] += jnp.dot(m, h2, preferred_element_type=jnp.float32)
    pcnt[...] += jnp.broadcast_to(jnp.sum(m, axis=1)[:, None], (G, H))

    @pl.when(i == NBLK - 1)
    def _():
        emb = psum[...] / jnp.maximum(pcnt[...], 1.0)
        z = jnp.maximum(
            jnp.dot(emb, wc1_ref[...], preferred_element_type=jnp.float32)
            + bc1_ref[...], 0.0)
        o_ref[0, :] = jnp.sum(z * wc2_ref[...], axis=1) + bc2_ref[0, 0]


def _tc_final(acc, cnt, h1, batch3, wl, wr, b, wc1, bc1, wc2, bc2):
    return pl.pallas_call(
        _tc_final_body,
        grid=(NBLK,),
        in_specs=[
            pl.BlockSpec((NC, R, D), lambda i: (0, i, 0)),
            pl.BlockSpec((NC, R, D), lambda i: (0, i, 0)),
            pl.BlockSpec((R, H), lambda i: (i, 0)),
            pl.BlockSpec((1, 1, R), lambda i: (i, 0, 0)),
            pl.BlockSpec((H, H), lambda i: (0, 0)),
            pl.BlockSpec((H, H), lambda i: (0, 0)),
            pl.BlockSpec((1, H), lambda i: (0, 0)),
            pl.BlockSpec((H, H), lambda i: (0, 0)),
            pl.BlockSpec((1, H), lambda i: (0, 0)),
            pl.BlockSpec((1, H), lambda i: (0, 0)),
            pl.BlockSpec((1, 1), lambda i: (0, 0)),
        ],
        out_specs=pl.BlockSpec((1, G), lambda i: (0, 0)),
        out_shape=jax.ShapeDtypeStruct((1, G), jnp.float32),
        scratch_shapes=[
            pltpu.VMEM((G, H), jnp.float32),
            pltpu.VMEM((G, H), jnp.float32),
        ],
    )(acc, cnt, h1, batch3, wl, wr, b, wc1, bc1, wc2, bc2)


def kernel(x, edge_index, batch, W_l0, b_l0, W_r0, W_l1, b_l1, W_r1,
           W_c1, b_c1, W_c2, b_c2):
    pad = EP - E
    ar = jnp.arange(pad, dtype=edge_index.dtype)
    pads = jnp.stack([(ar * 37) % N, TRASH + ar % (NP - N)])
    edges = jnp.concatenate([edge_index, pads], axis=1).reshape(2, NW * NCH, C)

    cnt, = _get_sc_deg()(edges)
    acc0, = _get_sc_agg()(x, edges)
    h1 = _tc_sage(acc0, cnt, x, W_l0, W_r0, b_l0.reshape(1, H))
    acc1, = _get_sc_agg()(h1, edges)
    out = _tc_final(acc1, cnt, h1, batch.reshape(NBLK, 1, R),
                    W_l1, W_r1, b_l1.reshape(1, H),
                    W_c1, b_c1.reshape(1, H), W_c2.reshape(1, H),
                    b_c2.reshape(1, 1))
    return out.reshape(G)
